# baseline (speedup 1.0000x reference)
import dataclasses
import functools

import jax
import jax.numpy as jnp
from jax import lax
from jax.experimental import pallas as pl
from jax.experimental.pallas import tpu as pltpu
from jax.experimental.pallas import tpu_sc as plsc

W = 64
NC = 2
NS = 16
NW = NC * NS
CHUNK = 80
GRP = 5

@functools.cache
def _sc_mesh():
    return plsc.VectorSubcoreMesh(core_axis_name="c", subcore_axis_name="s")

_no_layout_cp = pltpu.CompilerParams()
if "needs_layout_passes" in pltpu.CompilerParams.__dataclass_fields__:
    _no_layout_cp = dataclasses.replace(_no_layout_cp, needs_layout_passes=False)

_sc_flat_cp = pltpu.CompilerParams()
if "use_tc_tiling_on_sc" in pltpu.CompilerParams.__dataclass_fields__:
    _sc_flat_cp = dataclasses.replace(_sc_flat_cp, use_tc_tiling_on_sc=False)


def _wid():
    return lax.axis_index("s") * NC + lax.axis_index("c")


def _masked_src_idx(mask_i32, src, dst, n_nodes, dump_row):
    E = src.shape[0]
    e_pt = E // NW

    @functools.partial(
        pl.kernel,
        out_type=jax.ShapeDtypeStruct((E,), jnp.int32),
        mesh=_sc_mesh(),
        compiler_params=_no_layout_cp,
        scratch_types=[
            pltpu.VMEM((n_nodes,), jnp.int32),
            pltpu.VMEM((CHUNK,), jnp.int32),
            pltpu.VMEM((CHUNK,), jnp.int32),
            pltpu.VMEM((CHUNK,), jnp.int32),
        ],
    )
    def k(mask_hbm, src_hbm, dst_hbm, out_hbm, mask_v, s_v, d_v, o_v):
        base = _wid() * e_pt
        pltpu.sync_copy(mask_hbm, mask_v)

        @pl.loop(0, e_pt, step=CHUNK)
        def _(off):
            pltpu.sync_copy(src_hbm.at[pl.ds(base + off, CHUNK)], s_v)
            pltpu.sync_copy(dst_hbm.at[pl.ds(base + off, CHUNK)], d_v)

            @pl.loop(0, CHUNK, step=16)
            def _(i):
                si = s_v[pl.ds(i, 16)]
                di = d_v[pl.ds(i, 16)]
                ms = plsc.load_gather(mask_v, [si])
                md = plsc.load_gather(mask_v, [di])
                keep = (ms * md) > 0
                o_v[pl.ds(i, 16)] = jnp.where(keep, si, dump_row)

            pltpu.sync_copy(o_v, out_hbm.at[pl.ds(base + off, CHUNK)])

    return k(mask_i32, src, dst)


def _sc_gather2(a, b, dst, src):
    E = dst.shape[0]
    e_pt = E // NW
    out_sd = jax.ShapeDtypeStruct((E, W), jnp.float32)

    @functools.partial(
        pl.kernel,
        out_type=(out_sd, out_sd),
        mesh=_sc_mesh(),
        compiler_params=_sc_flat_cp,
        scratch_types=[
            pltpu.VMEM((GRP, CHUNK), jnp.int32),
            pltpu.VMEM((GRP, CHUNK), jnp.int32),
            pltpu.VMEM((GRP, CHUNK, W), jnp.float32),
            pltpu.VMEM((GRP, CHUNK, W), jnp.float32),
            pltpu.SemaphoreType.DMA,
            pltpu.SemaphoreType.DMA,
            pltpu.SemaphoreType.DMA,
        ],
    )
    def k(a_hbm, b_hbm, dst_hbm, src_hbm, g1_hbm, g2_hbm,
          di_v, si_v, ra_v, rb_v, sem_i, sem_g, sem_w):
        base = _wid() * e_pt

        @pl.loop(0, e_pt, step=GRP * CHUNK)
        def _(off):
            cs = []
            for j in range(GRP):
                o = base + off + j * CHUNK
                cs.append(pltpu.async_copy(
                    dst_hbm.at[pl.ds(o, CHUNK)], di_v.at[j], sem_i))
                cs.append(pltpu.async_copy(
                    src_hbm.at[pl.ds(o, CHUNK)], si_v.at[j], sem_i))
            for c in cs:
                c.wait()
            cs = []
            for j in range(GRP):
                cs.append(pltpu.async_copy(
                    a_hbm.at[di_v.at[j]], ra_v.at[j], sem_g))
                cs.append(pltpu.async_copy(
                    b_hbm.at[si_v.at[j]], rb_v.at[j], sem_g))
            for c in cs:
                c.wait()
            cs = []
            for j in range(GRP):
                o = base + off + j * CHUNK
                cs.append(pltpu.async_copy(
                    ra_v.at[j], g1_hbm.at[pl.ds(o, CHUNK)], sem_w))
                cs.append(pltpu.async_copy(
                    rb_v.at[j], g2_hbm.at[pl.ds(o, CHUNK)], sem_w))
            for c in cs:
                c.wait()

    return k(a, b, dst, src)


def _sc_scatter_add(ye, src_m, zeros_acc, n_acc):
    E = ye.shape[0]
    e_pt = E // NW
    rows_ps = n_acc // NS
    CH = CHUNK

    @functools.partial(
        pl.kernel,
        out_type=jax.ShapeDtypeStruct((NC, n_acc, W), jnp.float32),
        mesh=_sc_mesh(),
        compiler_params=_sc_flat_cp,
        scratch_types=[
            pltpu.VMEM((GRP, CH), jnp.int32),
            pltpu.VMEM((GRP, CH, W), jnp.float32),
            pltpu.VMEM_SHARED((n_acc, W), jnp.float32),
            pltpu.SemaphoreType.DMA,
            pltpu.SemaphoreType.DMA,
        ],
    )
    def k(ye_hbm, idx_hbm, z_hbm, out_hbm, i_v, r_v, acc, sem_l, sem_s):
        cid = lax.axis_index("c")
        sid = lax.axis_index("s")
        base = (sid * NC + cid) * e_pt
        r0 = sid * rows_ps
        pltpu.sync_copy(z_hbm.at[pl.ds(r0, rows_ps)], acc.at[pl.ds(r0, rows_ps)])
        plsc.subcore_barrier()

        @pl.loop(0, e_pt, step=GRP * CH)
        def _(off):
            cs = []
            for j in range(GRP):
                o = base + off + j * CH
                cs.append(pltpu.async_copy(
                    idx_hbm.at[pl.ds(o, CH)], i_v.at[j], sem_l))
                cs.append(pltpu.async_copy(
                    ye_hbm.at[pl.ds(o, CH)], r_v.at[j], sem_l))
            for c in cs:
                c.wait()
            cs = []
            for j in range(GRP):
                cs.append(pltpu.async_copy(
                    r_v.at[j], acc.at[i_v.at[j]], sem_s, add=True))
            for c in cs:
                c.wait()

        plsc.subcore_barrier()
        pltpu.sync_copy(acc.at[pl.ds(r0, rows_ps)],
                        out_hbm.at[cid, pl.ds(r0, rows_ps)])

    return k(ye, src_m, zeros_acc)


def _ln(t, g, b):
    m = jnp.mean(t, axis=-1, keepdims=True)
    v = jnp.mean((t - m) * (t - m), axis=-1, keepdims=True)
    return (t - m) * lax.rsqrt(v + 1e-5) * g + b


def _dot(a, b):
    return jnp.dot(a, b, preferred_element_type=jnp.float32)


def _full(shape):
    return pl.BlockSpec(shape, lambda i: tuple(0 for _ in shape))


def _rows(bm, width):
    return pl.BlockSpec((bm, width), lambda i: (i, 0))


def _tc_mlp_enc(x, p, bm, pad128=False):
    n, din = x.shape
    wo = 2 * W if pad128 else W

    def body(x_ref, w1_ref, b1_ref, w2_ref, b2_ref, g_ref, be_ref, o_ref):
        h = jnp.maximum(_dot(x_ref[...], w1_ref[...]) + b1_ref[...], 0.0)
        t = _dot(h, w2_ref[...]) + b2_ref[...]
        r = _ln(t, g_ref[...], be_ref[...])
        if pad128:
            r = jnp.concatenate([r, jnp.zeros_like(r)], axis=1)
        o_ref[...] = r

    return pl.pallas_call(
        body,
        grid=(n // bm,),
        in_specs=[
            _rows(bm, din),
            _full((din, W)), _full((1, W)),
            _full((W, W)), _full((1, W)),
            _full((1, W)), _full((1, W)),
        ],
        out_specs=_rows(bm, wo),
        out_shape=jax.ShapeDtypeStruct((n, wo), jnp.float32),
    )(x, p["W1"], p["b1"].reshape(1, W), p["W2"], p["b2"].reshape(1, W),
      p["g"].reshape(1, W), p["be"].reshape(1, W))


def _tc_ab(xn, w1i, w1j, b1):
    n = xn.shape[0]
    bm = 1000

    def body(x_ref, wi_ref, wj_ref, b1_ref, a_ref, b_ref):
        x = x_ref[...]
        a_ref[...] = _dot(x, wi_ref[...])
        b_ref[...] = _dot(x, wj_ref[...]) + b1_ref[...]

    sd = jax.ShapeDtypeStruct((n, W), jnp.float32)
    return pl.pallas_call(
        body,
        grid=(n // bm,),
        in_specs=[_rows(bm, W), _full((W, W)), _full((W, W)), _full((1, W))],
        out_specs=(_rows(bm, W), _rows(bm, W)),
        out_shape=(sd, sd),
    )(xn, w1i, w1j, b1.reshape(1, W))


def _tc_edge(ad, bs, xe, w1e, w2, b2, g, be, bm):
    e = xe.shape[0]

    def body(ad_ref, bs_ref, xe_ref, w1e_ref, w2_ref, b2_ref, g_ref, be_ref,
             o_ref):
        xe_b = xe_ref[...]
        h = jnp.maximum(ad_ref[...] + bs_ref[...] + _dot(xe_b, w1e_ref[...]),
                        0.0)
        t = _dot(h, w2_ref[...]) + b2_ref[...]
        o_ref[...] = xe_b + _ln(t, g_ref[...], be_ref[...])

    return pl.pallas_call(
        body,
        grid=(e // bm,),
        in_specs=[
            _rows(bm, W), _rows(bm, W), _rows(bm, W),
            _full((W, W)), _full((W, W)), _full((1, W)),
            _full((1, W)), _full((1, W)),
        ],
        out_specs=_rows(bm, W),
        out_shape=jax.ShapeDtypeStruct((e, W), jnp.float32),
    )(ad, bs, xe, w1e, w2, b2.reshape(1, W), g.reshape(1, W),
      be.reshape(1, W))


def _tc_node(msgs, xn, w1m, w1x, b1, w2, b2, g, be, bm):
    n = xn.shape[0]

    def body(ms_ref, xn_ref, wm_ref, wx_ref, b1_ref, w2_ref, b2_ref, g_ref,
             be_ref, o_ref):
        msg = ms_ref[0] + ms_ref[1]
        x = xn_ref[...]
        h = jnp.maximum(
            _dot(msg, wm_ref[...]) + _dot(x, wx_ref[...]) + b1_ref[...], 0.0)
        t = _dot(h, w2_ref[...]) + b2_ref[...]
        o_ref[...] = x + _ln(t, g_ref[...], be_ref[...])

    return pl.pallas_call(
        body,
        grid=(n // bm,),
        in_specs=[
            pl.BlockSpec((NC, bm, W), lambda i: (0, i, 0)),
            _rows(bm, W),
            _full((W, W)), _full((W, W)), _full((1, W)),
            _full((W, W)), _full((1, W)), _full((1, W)), _full((1, W)),
        ],
        out_specs=_rows(bm, W),
        out_shape=jax.ShapeDtypeStruct((n, W), jnp.float32),
    )(msgs, xn, w1m, w1x, b1.reshape(1, W), w2, b2.reshape(1, W),
      g.reshape(1, W), be.reshape(1, W))


def _tc_dec(xn, mask_f, d, bm):
    n = xn.shape[0]
    co = d["W2"].shape[1]

    def body(x_ref, m_ref, w1_ref, b1_ref, w2_ref, b2_ref, o_ref):
        h = jnp.maximum(_dot(x_ref[...], w1_ref[...]) + b1_ref[...], 0.0)
        o_ref[...] = (_dot(h, w2_ref[...]) + b2_ref[...]) * m_ref[...]

    return pl.pallas_call(
        body,
        grid=(n // bm,),
        in_specs=[
            _rows(bm, W), pl.BlockSpec((bm, 1), lambda i: (i, 0)),
            _full((W, W)), _full((1, W)),
            _full((W, co)), _full((1, co)),
        ],
        out_specs=_rows(bm, co),
        out_shape=jax.ShapeDtypeStruct((n, co), jnp.float32),
    )(xn, mask_f, d["W1"], d["b1"].reshape(1, W), d["W2"],
      d["b2"].reshape(1, co))


def kernel(x, edge_attr, params, edge_index, mask):
    n = x.shape[0]
    e = edge_attr.shape[0]
    bn = 1000
    be_blk = 3200
    n_acc = ((n + 1 + NS * 8 - 1) // (NS * 8)) * (NS * 8)

    src = edge_index[0]
    dst = edge_index[1]
    mask_i32 = mask.astype(jnp.int32)
    mask_f = mask.astype(jnp.float32).reshape(n, 1)
    zeros_acc = jnp.zeros((n_acc, W), jnp.float32)

    src_m = _masked_src_idx(mask_i32, src, dst, n, jnp.int32(n))

    xn = _tc_mlp_enc(x, params["node_enc"], bn)
    xe = _tc_mlp_enc(edge_attr, params["edge_enc"], be_blk)

    for lp in params["layers"]:
        ep = lp["edge"]
        np_ = lp["node"]
        w1i, w1j, w1e = ep["W1"][:W], ep["W1"][W:2 * W], ep["W1"][2 * W:]
        a, b = _tc_ab(xn, w1i, w1j, ep["b1"])
        ad, bs = _sc_gather2(a, b, dst, src)
        ye = _tc_edge(ad, bs, xe, w1e, ep["W2"], ep["b2"], ep["g"], ep["be"],
                      be_blk)
        msgs = _sc_scatter_add(ye, src_m, zeros_acc, n_acc)
        xn = _tc_node(msgs, xn, np_["W1"][:W], np_["W1"][W:], np_["b1"],
                      np_["W2"], np_["b2"], np_["g"], np_["be"], bn)
        xe = ye

    return _tc_dec(xn, mask_f, params["dec"], bn)


# speedup vs baseline: 1.3503x; 1.3503x over previous
import dataclasses
import functools

import jax
import jax.numpy as jnp
from jax import lax
from jax.experimental import pallas as pl
from jax.experimental.pallas import tpu as pltpu
from jax.experimental.pallas import tpu_sc as plsc

W = 64
NC = 2
NS = 16
NW = NC * NS
CHUNK = 80
GRP = 5

@functools.cache
def _sc_mesh():
    return plsc.VectorSubcoreMesh(core_axis_name="c", subcore_axis_name="s")

_no_layout_cp = pltpu.CompilerParams()
if "needs_layout_passes" in pltpu.CompilerParams.__dataclass_fields__:
    _no_layout_cp = dataclasses.replace(_no_layout_cp, needs_layout_passes=False)


def _wid():
    return lax.axis_index("s") * NC + lax.axis_index("c")


def _masked_src_idx(mask_i32, src, dst, n_nodes, dump_row):
    E = src.shape[0]
    e_pt = E // NW

    @functools.partial(
        pl.kernel,
        out_type=jax.ShapeDtypeStruct((E,), jnp.int32),
        mesh=_sc_mesh(),
        compiler_params=_no_layout_cp,
        scratch_types=[
            pltpu.VMEM((n_nodes,), jnp.int32),
            pltpu.VMEM((CHUNK,), jnp.int32),
            pltpu.VMEM((CHUNK,), jnp.int32),
            pltpu.VMEM((CHUNK,), jnp.int32),
        ],
    )
    def k(mask_hbm, src_hbm, dst_hbm, out_hbm, mask_v, s_v, d_v, o_v):
        base = _wid() * e_pt
        pltpu.sync_copy(mask_hbm, mask_v)

        @pl.loop(0, e_pt, step=CHUNK)
        def _(off):
            pltpu.sync_copy(src_hbm.at[pl.ds(base + off, CHUNK)], s_v)
            pltpu.sync_copy(dst_hbm.at[pl.ds(base + off, CHUNK)], d_v)

            @pl.loop(0, CHUNK, step=16)
            def _(i):
                si = s_v[pl.ds(i, 16)]
                di = d_v[pl.ds(i, 16)]
                ms = plsc.load_gather(mask_v, [si])
                md = plsc.load_gather(mask_v, [di])
                keep = (ms * md) > 0
                o_v[pl.ds(i, 16)] = jnp.where(keep, si, dump_row)

            pltpu.sync_copy(o_v, out_hbm.at[pl.ds(base + off, CHUNK)])

    return k(mask_i32, src, dst)


def _sc_gather2(tbl, dst, src):
    E = dst.shape[0]
    e_pt = E // NW
    out_sd = jax.ShapeDtypeStruct((E, 2 * W), jnp.float32)

    @functools.partial(
        pl.kernel,
        out_type=(out_sd, out_sd),
        mesh=_sc_mesh(),
        scratch_types=[
            pltpu.VMEM((GRP, CHUNK), jnp.int32),
            pltpu.VMEM((GRP, CHUNK), jnp.int32),
            pltpu.VMEM((GRP, CHUNK, 2 * W), jnp.float32),
            pltpu.VMEM((GRP, CHUNK, 2 * W), jnp.float32),
            pltpu.SemaphoreType.DMA,
            pltpu.SemaphoreType.DMA,
        ] + [pltpu.SemaphoreType.DMA] * GRP,
    )
    def k(t_hbm, dst_hbm, src_hbm, g1_hbm, g2_hbm,
          di_v, si_v, ra_v, rb_v, sem_i, sem_w, *gsems):
        base = _wid() * e_pt

        @pl.loop(0, e_pt, step=GRP * CHUNK)
        def _(off):
            cs = []
            for j in range(GRP):
                o = base + off + j * CHUNK
                cs.append(pltpu.async_copy(
                    dst_hbm.at[pl.ds(o, CHUNK)], di_v.at[j], sem_i))
                cs.append(pltpu.async_copy(
                    src_hbm.at[pl.ds(o, CHUNK)], si_v.at[j], sem_i))
            for c in cs:
                c.wait()
            cg = []
            for j in range(GRP):
                cg.append((
                    pltpu.async_copy(t_hbm.at[di_v.at[j]], ra_v.at[j],
                                     gsems[j]),
                    pltpu.async_copy(t_hbm.at[si_v.at[j]], rb_v.at[j],
                                     gsems[j]),
                ))
            cs = []
            for j in range(GRP):
                ca, cb = cg[j]
                ca.wait()
                cb.wait()
                o = base + off + j * CHUNK
                cs.append(pltpu.async_copy(
                    ra_v.at[j], g1_hbm.at[pl.ds(o, CHUNK)], sem_w))
                cs.append(pltpu.async_copy(
                    rb_v.at[j], g2_hbm.at[pl.ds(o, CHUNK)], sem_w))
            for c in cs:
                c.wait()

    return k(tbl, dst, src)


def _sc_scatter_add(ye, src_m, zeros_acc, n_acc):
    E = ye.shape[0]
    e_pt = E // NW
    rows_ps = n_acc // NS
    CH = CHUNK // 2

    @functools.partial(
        pl.kernel,
        out_type=jax.ShapeDtypeStruct((NC, n_acc, 2 * W), jnp.float32),
        mesh=_sc_mesh(),
        scratch_types=[
            pltpu.VMEM((GRP, CH), jnp.int32),
            pltpu.VMEM((GRP, CH, 2 * W), jnp.float32),
            pltpu.VMEM_SHARED((n_acc, 2 * W), jnp.float32),
            pltpu.SemaphoreType.DMA,
        ] + [pltpu.SemaphoreType.DMA] * GRP,
    )
    def k(ye_hbm, idx_hbm, z_hbm, out_hbm, i_v, r_v, acc, sem_s, *lsems):
        cid = lax.axis_index("c")
        sid = lax.axis_index("s")
        base = (sid * NC + cid) * e_pt
        r0 = sid * rows_ps
        pltpu.sync_copy(z_hbm.at[pl.ds(r0, rows_ps)], acc.at[pl.ds(r0, rows_ps)])
        plsc.subcore_barrier()

        @pl.loop(0, e_pt, step=GRP * CH)
        def _(off):
            cl = []
            for j in range(GRP):
                o = base + off + j * CH
                cl.append((
                    pltpu.async_copy(idx_hbm.at[pl.ds(o, CH)], i_v.at[j],
                                     lsems[j]),
                    pltpu.async_copy(ye_hbm.at[pl.ds(o, CH)], r_v.at[j],
                                     lsems[j]),
                ))
            cs = []
            for j in range(GRP):
                ca, cb = cl[j]
                ca.wait()
                cb.wait()
                cs.append(pltpu.async_copy(
                    r_v.at[j], acc.at[i_v.at[j]], sem_s, add=True))
            for c in cs:
                c.wait()

        plsc.subcore_barrier()
        pltpu.sync_copy(acc.at[pl.ds(r0, rows_ps)],
                        out_hbm.at[cid, pl.ds(r0, rows_ps)])

    return k(ye, src_m, zeros_acc)


def _ln(t, g, b):
    m = jnp.mean(t, axis=-1, keepdims=True)
    v = jnp.mean((t - m) * (t - m), axis=-1, keepdims=True)
    return (t - m) * lax.rsqrt(v + 1e-5) * g + b


def _dot(a, b):
    return jnp.dot(a, b, preferred_element_type=jnp.float32)


def _full(shape):
    return pl.BlockSpec(shape, lambda i: tuple(0 for _ in shape))


def _rows(bm, width):
    return pl.BlockSpec((bm, width), lambda i: (i, 0))


def _tc_mlp_enc(x, p, bm, pad128=False):
    n, din = x.shape
    wo = 2 * W if pad128 else W

    def body(x_ref, w1_ref, b1_ref, w2_ref, b2_ref, g_ref, be_ref, o_ref):
        h = jnp.maximum(_dot(x_ref[...], w1_ref[...]) + b1_ref[...], 0.0)
        t = _dot(h, w2_ref[...]) + b2_ref[...]
        r = _ln(t, g_ref[...], be_ref[...])
        if pad128:
            r = jnp.concatenate([r, jnp.zeros_like(r)], axis=1)
        o_ref[...] = r

    return pl.pallas_call(
        body,
        grid=(n // bm,),
        in_specs=[
            _rows(bm, din),
            _full((din, W)), _full((1, W)),
            _full((W, W)), _full((1, W)),
            _full((1, W)), _full((1, W)),
        ],
        out_specs=_rows(bm, wo),
        out_shape=jax.ShapeDtypeStruct((n, wo), jnp.float32),
    )(x, p["W1"], p["b1"].reshape(1, W), p["W2"], p["b2"].reshape(1, W),
      p["g"].reshape(1, W), p["be"].reshape(1, W))


def _tc_ab(xn, w1i, w1j, b1):
    n = xn.shape[0]
    bm = 1000

    def body(x_ref, wi_ref, wj_ref, b1_ref, t_ref):
        x = x_ref[...]
        a = _dot(x, wi_ref[...])
        b = _dot(x, wj_ref[...]) + b1_ref[...]
        t_ref[...] = jnp.concatenate([a, b], axis=1)

    return pl.pallas_call(
        body,
        grid=(n // bm,),
        in_specs=[_rows(bm, W), _full((W, W)), _full((W, W)), _full((1, W))],
        out_specs=_rows(bm, 2 * W),
        out_shape=jax.ShapeDtypeStruct((n, 2 * W), jnp.float32),
    )(xn, w1i, w1j, b1.reshape(1, W))


def _tc_edge(g1, g2, xe, w1e, w2, b2, g, be, bm):
    e = xe.shape[0]

    def body(g1_ref, g2_ref, xe_ref, w1e_ref, w2_ref, b2_ref, g_ref, be_ref,
             o_ref):
        xe_b = xe_ref[:, :W]
        ad = g1_ref[:, :W].astype(jnp.float32)
        bs = g2_ref[:, W:].astype(jnp.float32)
        h = jnp.maximum(ad + bs + _dot(xe_b, w1e_ref[...]), 0.0)
        t = _dot(h, w2_ref[...]) + b2_ref[...]
        r = xe_b + _ln(t, g_ref[...], be_ref[...])
        o_ref[...] = jnp.concatenate([r, jnp.zeros_like(r)], axis=1)

    return pl.pallas_call(
        body,
        grid=(e // bm,),
        in_specs=[
            _rows(bm, 2 * W), _rows(bm, 2 * W), _rows(bm, 2 * W),
            _full((W, W)), _full((W, W)), _full((1, W)),
            _full((1, W)), _full((1, W)),
        ],
        out_specs=_rows(bm, 2 * W),
        out_shape=jax.ShapeDtypeStruct((e, 2 * W), jnp.float32),
    )(g1, g2, xe, w1e, w2, b2.reshape(1, W), g.reshape(1, W),
      be.reshape(1, W))


def _tc_node(msgs, xn, w1m, w1x, b1, w2, b2, g, be, bm):
    n = xn.shape[0]

    def body(ms_ref, xn_ref, wm_ref, wx_ref, b1_ref, w2_ref, b2_ref, g_ref,
             be_ref, o_ref):
        msg = ms_ref[0][:, :W] + ms_ref[1][:, :W]
        x = xn_ref[...]
        h = jnp.maximum(
            _dot(msg, wm_ref[...]) + _dot(x, wx_ref[...]) + b1_ref[...], 0.0)
        t = _dot(h, w2_ref[...]) + b2_ref[...]
        o_ref[...] = x + _ln(t, g_ref[...], be_ref[...])

    return pl.pallas_call(
        body,
        grid=(n // bm,),
        in_specs=[
            pl.BlockSpec((NC, bm, 2 * W), lambda i: (0, i, 0)),
            _rows(bm, W),
            _full((W, W)), _full((W, W)), _full((1, W)),
            _full((W, W)), _full((1, W)), _full((1, W)), _full((1, W)),
        ],
        out_specs=_rows(bm, W),
        out_shape=jax.ShapeDtypeStruct((n, W), jnp.float32),
    )(msgs, xn, w1m, w1x, b1.reshape(1, W), w2, b2.reshape(1, W),
      g.reshape(1, W), be.reshape(1, W))


def _tc_dec(xn, mask_f, d, bm):
    n = xn.shape[0]
    co = d["W2"].shape[1]

    def body(x_ref, m_ref, w1_ref, b1_ref, w2_ref, b2_ref, o_ref):
        h = jnp.maximum(_dot(x_ref[...], w1_ref[...]) + b1_ref[...], 0.0)
        o_ref[...] = (_dot(h, w2_ref[...]) + b2_ref[...]) * m_ref[...]

    return pl.pallas_call(
        body,
        grid=(n // bm,),
        in_specs=[
            _rows(bm, W), pl.BlockSpec((bm, 1), lambda i: (i, 0)),
            _full((W, W)), _full((1, W)),
            _full((W, co)), _full((1, co)),
        ],
        out_specs=_rows(bm, co),
        out_shape=jax.ShapeDtypeStruct((n, co), jnp.float32),
    )(xn, mask_f, d["W1"], d["b1"].reshape(1, W), d["W2"],
      d["b2"].reshape(1, co))


def kernel(x, edge_attr, params, edge_index, mask):
    n = x.shape[0]
    e = edge_attr.shape[0]
    bn = 1000
    be_blk = 3200
    n_acc = ((n + 1 + NS * 8 - 1) // (NS * 8)) * (NS * 8)

    src = edge_index[0]
    dst = edge_index[1]
    mask_i32 = mask.astype(jnp.int32)
    mask_f = mask.astype(jnp.float32).reshape(n, 1)
    zeros_acc = jnp.zeros((n_acc, 2 * W), jnp.float32)

    src_m = _masked_src_idx(mask_i32, src, dst, n, jnp.int32(n))

    xn = _tc_mlp_enc(x, params["node_enc"], bn)
    xe = _tc_mlp_enc(edge_attr, params["edge_enc"], be_blk, pad128=True)

    for lp in params["layers"]:
        ep = lp["edge"]
        np_ = lp["node"]
        w1i, w1j, w1e = ep["W1"][:W], ep["W1"][W:2 * W], ep["W1"][2 * W:]
        tbl = _tc_ab(xn, w1i, w1j, ep["b1"])
        g1, g2 = _sc_gather2(tbl, dst, src)
        ye = _tc_edge(g1, g2, xe, w1e, ep["W2"], ep["b2"], ep["g"], ep["be"],
                      be_blk)
        msgs = _sc_scatter_add(ye, src_m, zeros_acc, n_acc)
        xn = _tc_node(msgs, xn, np_["W1"][:W], np_["W1"][W:], np_["b1"],
                      np_["W2"], np_["b2"], np_["g"], np_["be"], bn)
        xe = ye

    return _tc_dec(xn, mask_f, params["dec"], bn)
